# baseline (speedup 1.0000x reference)
import functools

import jax
import jax.numpy as jnp
from jax import lax
from jax.experimental import pallas as pl
from jax.experimental.pallas import tpu as pltpu
from jax.experimental.pallas import tpu_sc as plsc

_N = 50000
_E = 800000
_H = 4
_C = 8
_HC = _H * _C
_IN0 = 16
_OUT = 32
_ROW = 40

_BN = 2000
_NC = 2
_NS = 16
_RPT = 3136
_RPT_LAST = _N - (_NS - 1) * _RPT
_EHALF = _E // _NC
_EPT = _EHALF // _NS
_B = 40
_NCHUNK = _EPT // _B

_F32 = jnp.float32


def _dyngather(x, idx):
    dnums = lax.GatherDimensionNumbers(
        offset_dims=(), collapsed_slice_dims=(0,), start_index_map=(0,))
    return lax.gather(x, idx[:, None], dnums, (1,),
                      mode=lax.GatherScatterMode.PROMISE_IN_BOUNDS)


def _sc_edge_pass(src_hbm, dst_hbm, xl_hbm, xr_hbm, att_hbm, init_hbm,
                  out_hbm,
                  acc, srcv, dstv, fbuf, rows_r, att_v,
                  si, sgl, sgr, ss):
    c = lax.axis_index("c")
    s = lax.axis_index("s")

    @pl.when(s < _NS - 1)
    def _():
        pltpu.sync_copy(init_hbm.at[c, pl.ds(s * _RPT, _RPT)],
                        acc.at[pl.ds(s * _RPT, _RPT)])

    @pl.when(s == _NS - 1)
    def _():
        pltpu.sync_copy(init_hbm.at[c, pl.ds((_NS - 1) * _RPT, _RPT_LAST)],
                        acc.at[pl.ds((_NS - 1) * _RPT, _RPT_LAST)])

    pltpu.sync_copy(att_hbm, att_v)
    plsc.subcore_barrier()

    iota = lax.iota(jnp.int32, 16)
    bfly1 = jnp.bitwise_xor(iota, 1)
    bfly2 = jnp.bitwise_xor(iota, 2)
    bfly4 = jnp.bitwise_xor(iota, 4)
    idx_hi = jnp.bitwise_and(iota, 7) + 8
    idx9 = jnp.where(iota == 9, 8, 0)
    idx11 = jnp.where(iota == 11, 8, 0)
    lt8 = iota < 8
    lt10 = iota < 10
    lt12 = iota < 12
    zf = jnp.zeros((16,), _F32)
    attA = att_v[0:16]
    attB = att_v[16:32]

    ebase = c * _EHALF + s * _EPT

    def idx_copies(g, slot):
        base = ebase + g * _B
        return (pltpu.make_async_copy(src_hbm.at[pl.ds(base, _B)],
                                      srcv.at[slot], si.at[slot]),
                pltpu.make_async_copy(dst_hbm.at[pl.ds(base, _B)],
                                      dstv.at[slot], si.at[slot]))

    def gather_copies(slot2, slot3):
        return (pltpu.make_async_copy(xl_hbm.at[srcv.at[slot3]],
                                      fbuf.at[slot2], sgl.at[slot2]),
                pltpu.make_async_copy(xr_hbm.at[dstv.at[slot3]],
                                      rows_r.at[slot2], sgr.at[slot2]))

    def scatter_copy(slot2, slot3):
        return pltpu.make_async_copy(fbuf.at[slot2],
                                     acc.at[dstv.at[slot3]],
                                     ss.at[slot2])

    def head_sums(t):
        t = t + _dyngather(t, bfly1)
        t = t + _dyngather(t, bfly2)
        return t + _dyngather(t, bfly4)

    def compute_chunk(slot):
        def edge_body(b, _):
            vlA = fbuf[slot, b, 0:16]
            vlB = fbuf[slot, b, 16:32]
            vrA = rows_r[slot, b, 0:16]
            vrB = rows_r[slot, b, 16:32]
            sA = vlA + vrA
            sB = vlB + vrB
            mA = jnp.maximum(sA, 0.2 * sA)
            mB = jnp.maximum(sB, 0.2 * sB)
            eA = jnp.exp(head_sums(mA * attA))
            eB = jnp.exp(head_sums(mB * attB))
            numA = eA * vlA
            numB = eB * vlB
            fbuf[slot, b, 0:16] = numA
            fbuf[slot, b, 16:32] = numB
            hiB = _dyngather(numB, idx_hi)
            gA3 = _dyngather(eA, idx9)
            gB3 = _dyngather(eB, idx11)
            tail = jnp.where(lt8, hiB,
                   jnp.where(lt10, gA3,
                   jnp.where(lt12, gB3, zf)))
            fbuf[slot, b, 24:40] = tail
            return 0

        lax.fori_loop(0, _B, edge_body, 0, unroll=8)

    ca0, cb0 = idx_copies(0, 0)
    ca0.start()
    cb0.start()

    def macro_body(p, _):
        for k6 in range(6):
            g = 6 * p + k6
            s2 = k6 % 2
            s2c = (k6 + 1) % 2
            s3 = k6 % 3
            s3n = (k6 + 1) % 3
            s3c = (k6 + 2) % 3

            @pl.when(jnp.logical_and(g >= 2, g <= _NCHUNK + 1))
            def _():
                scatter_copy(s2, s3n).wait()

            @pl.when(g <= _NCHUNK - 2)
            def _():
                ca, cb = idx_copies(g + 1, s3n)
                ca.start()
                cb.start()

            @pl.when(g <= _NCHUNK - 1)
            def _():
                ca, cb = idx_copies(g, s3)
                ca.wait()
                cb.wait()
                cl, cr = gather_copies(s2, s3)
                cl.start()
                cr.start()

            @pl.when(jnp.logical_and(g >= 1, g <= _NCHUNK))
            def _():
                cl, cr = gather_copies(s2c, s3c)
                cl.wait()
                cr.wait()
                compute_chunk(s2c)
                scatter_copy(s2c, s3c).start(add=True)
        return 0

    lax.fori_loop(0, (_NCHUNK + 7) // 6, macro_body, 0)
    plsc.subcore_barrier()

    @pl.when(s < _NS - 1)
    def _():
        pltpu.sync_copy(acc.at[pl.ds(s * _RPT, _RPT)],
                        out_hbm.at[c, pl.ds(s * _RPT, _RPT)])

    @pl.when(s == _NS - 1)
    def _():
        pltpu.sync_copy(acc.at[pl.ds((_NS - 1) * _RPT, _RPT_LAST)],
                        out_hbm.at[c, pl.ds((_NS - 1) * _RPT, _RPT_LAST)])


_sc_call = pl.kernel(
    _sc_edge_pass,
    out_type=[jax.ShapeDtypeStruct((_NC, _N, _ROW), _F32)],
    mesh=plsc.VectorSubcoreMesh(core_axis_name="c", subcore_axis_name="s"),
    compiler_params=pltpu.CompilerParams(use_tc_tiling_on_sc=False),
    scratch_types=[
        pltpu.VMEM_SHARED((_N, _ROW), _F32),
        pltpu.VMEM((3, _B), jnp.int32),
        pltpu.VMEM((3, _B), jnp.int32),
        pltpu.VMEM((2, _B, _ROW), _F32),
        pltpu.VMEM((2, _B, _HC), _F32),
        pltpu.VMEM((_HC,), _F32),
        pltpu.SemaphoreType.DMA((3,)),
        pltpu.SemaphoreType.DMA((2,)),
        pltpu.SemaphoreType.DMA((2,)),
        pltpu.SemaphoreType.DMA((2,)),
    ],
)


def _mm(a, b):
    return jnp.dot(a, b, preferred_element_type=_F32,
                   precision=lax.Precision.HIGHEST)


def _ln(h, g, b):
    mu = jnp.mean(h, axis=1, keepdims=True)
    var = jnp.mean((h - mu) ** 2, axis=1, keepdims=True)
    return (h - mu) * lax.rsqrt(var + 1e-5) * g + b


def _elu(h):
    return jnp.where(h > 0, h, jnp.exp(jnp.minimum(h, 0.0)) - 1.0)


def _self_terms(xl, xr, aatt, rrep):
    s = xl + xr
    m = jnp.maximum(s, 0.2 * s)
    e = jnp.exp(_mm(m, aatt))
    sn = _mm(e, rrep) * xl
    return sn, e


def _store_init(xl, xr, sn, e, xl_ref, xr_ref, init_ref):
    xl_ref[...] = jnp.concatenate(
        [xl, jnp.zeros((_BN, _ROW - _HC), _F32)], axis=1)
    xr_ref[...] = xr
    z4 = jnp.zeros((_BN, _ROW - _HC - _H), _F32)
    init_ref[0] = jnp.concatenate([sn, e, z4], axis=1)
    init_ref[1] = jnp.zeros((_BN, _ROW), _F32)


def _pre_body(x_ref, jt_ref, emb_ref, wl_ref, wr_ref, aatt_ref, rrep_ref,
              xl_ref, xr_ref, init_ref):
    jt = jt_ref[...]
    oh = (jt == lax.broadcasted_iota(jnp.int32, (_BN, 17), 1)).astype(_F32)
    je = _mm(oh, emb_ref[...])
    h = jnp.concatenate([x_ref[...], je], axis=1)
    xl = _mm(h, wl_ref[...])
    xr = _mm(h, wr_ref[...])
    sn, e = _self_terms(xl, xr, aatt_ref[...], rrep_ref[...])
    _store_init(xl, xr, sn, e, xl_ref, xr_ref, init_ref)


def _combine(acc_ref, rrep):
    num = acc_ref[0][:, 0:_HC] + acc_ref[1][:, 0:_HC]
    den4 = (acc_ref[0][:, _HC:_HC + _H]
            + acc_ref[1][:, _HC:_HC + _H])
    return num / _mm(den4, rrep)


def _mid_body(acc_ref, g_ref, be_ref, b_ref,
              wl_ref, wr_ref, aatt_ref, rrep_ref,
              xl_ref, xr_ref, init_ref):
    rrep = rrep_ref[...]
    h = _combine(acc_ref, rrep)
    h = _elu(_ln(h + b_ref[...], g_ref[...], be_ref[...]))
    xl = _mm(h, wl_ref[...])
    xr = _mm(h, wr_ref[...])
    sn, e = _self_terms(xl, xr, aatt_ref[...], rrep)
    _store_init(xl, xr, sn, e, xl_ref, xr_ref, init_ref)


def _post_body(acc_ref, mh_ref, rrep_ref,
               b1_ref, g1_ref, be1_ref, wp_ref, bp_ref, gf_ref, bf_ref,
               out_ref):
    outm = _combine(acc_ref, rrep_ref[...])
    h = _mm(outm, mh_ref[...])
    h = _elu(_ln(h + b1_ref[...], g1_ref[...], be1_ref[...]))
    emb = _mm(h, wp_ref[...]) + bp_ref[...]
    emb = _ln(emb, gf_ref[...], bf_ref[...])
    nrm = jnp.sqrt(jnp.sum(emb * emb, axis=1, keepdims=True))
    out_ref[...] = emb / jnp.maximum(nrm, 1e-12)


def _row_spec(cols):
    return pl.BlockSpec((_BN, cols), lambda i: (i, 0))


def _acc_spec():
    return pl.BlockSpec((_NC, _BN, _ROW), lambda i: (0, i, 0))


def _full_spec(shape):
    return pl.BlockSpec(shape, lambda i: tuple(0 for _ in shape))


_GRID = (_N // _BN,)

_STAGE_OUT = [jax.ShapeDtypeStruct((_N, _ROW), _F32),
              jax.ShapeDtypeStruct((_N, _HC), _F32),
              jax.ShapeDtypeStruct((_NC, _N, _ROW), _F32)]
_STAGE_OUT_SPECS = [_row_spec(_ROW), _row_spec(_HC), _acc_spec()]

_pre_call = pl.pallas_call(
    _pre_body,
    grid=_GRID,
    in_specs=[_row_spec(4), _row_spec(1), _full_spec((17, 12)),
              _full_spec((_IN0, _HC)), _full_spec((_IN0, _HC)),
              _full_spec((_HC, _H)), _full_spec((_H, _HC))],
    out_specs=_STAGE_OUT_SPECS,
    out_shape=_STAGE_OUT,
)

_mid_call = pl.pallas_call(
    _mid_body,
    grid=_GRID,
    in_specs=[_acc_spec(),
              _full_spec((1, _HC)), _full_spec((1, _HC)), _full_spec((1, _HC)),
              _full_spec((_HC, _HC)), _full_spec((_HC, _HC)),
              _full_spec((_HC, _H)), _full_spec((_H, _HC))],
    out_specs=_STAGE_OUT_SPECS,
    out_shape=_STAGE_OUT,
)

_post_call = pl.pallas_call(
    _post_body,
    grid=_GRID,
    in_specs=[_acc_spec(),
              _full_spec((_HC, _C)), _full_spec((_H, _HC)),
              _full_spec((1, _C)), _full_spec((1, _C)), _full_spec((1, _C)),
              _full_spec((_C, _OUT)), _full_spec((1, _OUT)),
              _full_spec((1, _OUT)), _full_spec((1, _OUT))],
    out_specs=[_row_spec(_OUT)],
    out_shape=[jax.ShapeDtypeStruct((_N, _OUT), _F32)],
)


def _att_mat(att):
    return (att[:, :, None] * jnp.eye(_H, dtype=_F32)[:, None, :]).reshape(_HC, _H)


def kernel(x, joint_types, edge_index, emb_table, Wl0, Wr0, att0, b0, g0, be0,
           Wl1, Wr1, att1, b1, g1, be1, Wp, bp, gf, bf):
    jt = joint_types.astype(jnp.int32).reshape(_N, 1)
    src = edge_index[0].astype(jnp.int32)
    dst = edge_index[1].astype(jnp.int32)

    rrep = jnp.repeat(jnp.eye(_H, dtype=_F32), _C, axis=1)
    mh = jnp.tile(jnp.eye(_C, dtype=_F32), (_H, 1)) * (1.0 / _H)
    aatt0 = _att_mat(att0)
    aatt1 = _att_mat(att1)
    attv0 = att0.reshape(_HC)
    attv1 = att1.reshape(_HC)

    xl0, xr0, init0 = _pre_call(x, jt, emb_table, Wl0, Wr0, aatt0, rrep)
    (acc0,) = _sc_call(src, dst, xl0, xr0, attv0, init0)

    xl1, xr1, init1 = _mid_call(acc0,
                                g0.reshape(1, _HC), be0.reshape(1, _HC),
                                b0.reshape(1, _HC), Wl1, Wr1, aatt1, rrep)
    (acc1,) = _sc_call(src, dst, xl1, xr1, attv1, init1)

    (out,) = _post_call(acc1, mh, rrep,
                        b1.reshape(1, _C), g1.reshape(1, _C),
                        be1.reshape(1, _C), Wp, bp.reshape(1, _OUT),
                        gf.reshape(1, _OUT), bf.reshape(1, _OUT))
    return out


# speedup vs baseline: 1.2153x; 1.2153x over previous
import functools

import jax
import jax.numpy as jnp
from jax import lax
from jax.experimental import pallas as pl
from jax.experimental.pallas import tpu as pltpu
from jax.experimental.pallas import tpu_sc as plsc

_N = 50000
_E = 800000
_H = 4
_C = 8
_HC = _H * _C
_IN0 = 16
_OUT = 32
_ROW = 40

_BN = 2000
_NC = 2
_NS = 16
_RPT = 3136
_RPT_LAST = _N - (_NS - 1) * _RPT
_EHALF = _E // _NC
_EPT = _EHALF // _NS
_B = 40
_NCHUNK = _EPT // _B

_F32 = jnp.float32


def _dyngather(x, idx):
    dnums = lax.GatherDimensionNumbers(
        offset_dims=(), collapsed_slice_dims=(0,), start_index_map=(0,))
    return lax.gather(x, idx[:, None], dnums, (1,),
                      mode=lax.GatherScatterMode.PROMISE_IN_BOUNDS)


def _sc_edge_pass(src_hbm, dst_hbm, xl_hbm, xr_hbm, att_hbm, init_hbm,
                  out_hbm,
                  acc, srcv, dstv, fbuf, rows_r, att_v,
                  si, sgl, sgr, ss):
    c = lax.axis_index("c")
    s = lax.axis_index("s")

    @pl.when(s < _NS - 1)
    def _():
        pltpu.sync_copy(init_hbm.at[c, pl.ds(s * _RPT, _RPT)],
                        acc.at[pl.ds(s * _RPT, _RPT)])

    @pl.when(s == _NS - 1)
    def _():
        pltpu.sync_copy(init_hbm.at[c, pl.ds((_NS - 1) * _RPT, _RPT_LAST)],
                        acc.at[pl.ds((_NS - 1) * _RPT, _RPT_LAST)])

    pltpu.sync_copy(att_hbm, att_v)
    plsc.subcore_barrier()

    iota = lax.iota(jnp.int32, 16)
    bfly1 = jnp.bitwise_xor(iota, 1)
    bfly2 = jnp.bitwise_xor(iota, 2)
    bfly4 = jnp.bitwise_xor(iota, 4)
    idx_hi = jnp.bitwise_and(iota, 7) + 8
    idx9 = jnp.where(iota == 9, 8, 0)
    idx11 = jnp.where(iota == 11, 8, 0)
    lt8 = iota < 8
    lt10 = iota < 10
    lt12 = iota < 12
    zf = jnp.zeros((16,), _F32)
    attA = att_v[0:16]
    attB = att_v[16:32]

    ebase = c * _EHALF + s * _EPT

    def idx_copies(g, slot):
        base = ebase + g * _B
        return (pltpu.make_async_copy(src_hbm.at[pl.ds(base, _B)],
                                      srcv.at[slot], si.at[slot]),
                pltpu.make_async_copy(dst_hbm.at[pl.ds(base, _B)],
                                      dstv.at[slot], si.at[slot]))

    def gather_copies(slot2, slot3):
        return (pltpu.make_async_copy(xl_hbm.at[srcv.at[slot3]],
                                      fbuf.at[slot2], sgl.at[slot2]),
                pltpu.make_async_copy(xr_hbm.at[dstv.at[slot3]],
                                      rows_r.at[slot2], sgr.at[slot2]))

    def scatter_copy(slot2, slot3):
        return pltpu.make_async_copy(fbuf.at[slot2],
                                     acc.at[dstv.at[slot3]],
                                     ss.at[slot2])

    def head_sums(t):
        t = t + _dyngather(t, bfly1)
        t = t + _dyngather(t, bfly2)
        return t + _dyngather(t, bfly4)

    def compute_chunk(slot):
        def edge_body(b, _):
            vlA = fbuf[slot, b, 0:16]
            vlB = fbuf[slot, b, 16:32]
            vrA = rows_r[slot, b, 0:16]
            vrB = rows_r[slot, b, 16:32]
            sA = vlA + vrA
            sB = vlB + vrB
            mA = jnp.maximum(sA, 0.2 * sA)
            mB = jnp.maximum(sB, 0.2 * sB)
            eA = jnp.exp(head_sums(mA * attA))
            eB = jnp.exp(head_sums(mB * attB))
            numA = eA * vlA
            numB = eB * vlB
            fbuf[slot, b, 0:16] = numA
            fbuf[slot, b, 16:32] = numB
            hiB = _dyngather(numB, idx_hi)
            gA3 = _dyngather(eA, idx9)
            gB3 = _dyngather(eB, idx11)
            tail = jnp.where(lt8, hiB,
                   jnp.where(lt10, gA3,
                   jnp.where(lt12, gB3, zf)))
            fbuf[slot, b, 24:40] = tail
            return 0

        lax.fori_loop(0, _B, edge_body, 0, unroll=2)

    ca0, cb0 = idx_copies(0, 0)
    ca0.start()
    cb0.start()

    def macro_body(p, _):
        for k6 in range(6):
            g = 6 * p + k6
            s2 = k6 % 2
            s2c = (k6 + 1) % 2
            s3 = k6 % 3
            s3n = (k6 + 1) % 3
            s3c = (k6 + 2) % 3

            @pl.when(jnp.logical_and(g >= 2, g <= _NCHUNK + 1))
            def _():
                scatter_copy(s2, s3n).wait()

            @pl.when(g <= _NCHUNK - 2)
            def _():
                ca, cb = idx_copies(g + 1, s3n)
                ca.start()
                cb.start()

            @pl.when(g <= _NCHUNK - 1)
            def _():
                ca, cb = idx_copies(g, s3)
                ca.wait()
                cb.wait()
                cl, cr = gather_copies(s2, s3)
                cl.start()
                cr.start()

            @pl.when(jnp.logical_and(g >= 1, g <= _NCHUNK))
            def _():
                cl, cr = gather_copies(s2c, s3c)
                cl.wait()
                cr.wait()
                compute_chunk(s2c)
                scatter_copy(s2c, s3c).start(add=True)
        return 0

    lax.fori_loop(0, (_NCHUNK + 7) // 6, macro_body, 0)
    plsc.subcore_barrier()

    @pl.when(s < _NS - 1)
    def _():
        pltpu.sync_copy(acc.at[pl.ds(s * _RPT, _RPT)],
                        out_hbm.at[c, pl.ds(s * _RPT, _RPT)])

    @pl.when(s == _NS - 1)
    def _():
        pltpu.sync_copy(acc.at[pl.ds((_NS - 1) * _RPT, _RPT_LAST)],
                        out_hbm.at[c, pl.ds((_NS - 1) * _RPT, _RPT_LAST)])


_sc_call = pl.kernel(
    _sc_edge_pass,
    out_type=[jax.ShapeDtypeStruct((_NC, _N, _ROW), _F32)],
    mesh=plsc.VectorSubcoreMesh(core_axis_name="c", subcore_axis_name="s"),
    compiler_params=pltpu.CompilerParams(use_tc_tiling_on_sc=False),
    scratch_types=[
        pltpu.VMEM_SHARED((_N, _ROW), _F32),
        pltpu.VMEM((3, _B), jnp.int32),
        pltpu.VMEM((3, _B), jnp.int32),
        pltpu.VMEM((2, _B, _ROW), _F32),
        pltpu.VMEM((2, _B, _HC), _F32),
        pltpu.VMEM((_HC,), _F32),
        pltpu.SemaphoreType.DMA((3,)),
        pltpu.SemaphoreType.DMA((2,)),
        pltpu.SemaphoreType.DMA((2,)),
        pltpu.SemaphoreType.DMA((2,)),
    ],
)


def _mm(a, b):
    return jnp.dot(a, b, preferred_element_type=_F32,
                   precision=lax.Precision.HIGHEST)


def _ln(h, g, b):
    mu = jnp.mean(h, axis=1, keepdims=True)
    var = jnp.mean((h - mu) ** 2, axis=1, keepdims=True)
    return (h - mu) * lax.rsqrt(var + 1e-5) * g + b


def _elu(h):
    return jnp.where(h > 0, h, jnp.exp(jnp.minimum(h, 0.0)) - 1.0)


def _self_terms(xl, xr, aatt, rrep):
    s = xl + xr
    m = jnp.maximum(s, 0.2 * s)
    e = jnp.exp(_mm(m, aatt))
    sn = _mm(e, rrep) * xl
    return sn, e


def _store_init(xl, xr, sn, e, xl_ref, xr_ref, init_ref):
    xl_ref[...] = jnp.concatenate(
        [xl, jnp.zeros((_BN, _ROW - _HC), _F32)], axis=1)
    xr_ref[...] = xr
    z4 = jnp.zeros((_BN, _ROW - _HC - _H), _F32)
    init_ref[0] = jnp.concatenate([sn, e, z4], axis=1)
    init_ref[1] = jnp.zeros((_BN, _ROW), _F32)


def _pre_body(x_ref, jt_ref, emb_ref, wl_ref, wr_ref, aatt_ref, rrep_ref,
              xl_ref, xr_ref, init_ref):
    jt = jt_ref[...]
    oh = (jt == lax.broadcasted_iota(jnp.int32, (_BN, 17), 1)).astype(_F32)
    je = _mm(oh, emb_ref[...])
    h = jnp.concatenate([x_ref[...], je], axis=1)
    xl = _mm(h, wl_ref[...])
    xr = _mm(h, wr_ref[...])
    sn, e = _self_terms(xl, xr, aatt_ref[...], rrep_ref[...])
    _store_init(xl, xr, sn, e, xl_ref, xr_ref, init_ref)


def _combine(acc_ref, rrep):
    num = acc_ref[0][:, 0:_HC] + acc_ref[1][:, 0:_HC]
    den4 = (acc_ref[0][:, _HC:_HC + _H]
            + acc_ref[1][:, _HC:_HC + _H])
    return num / _mm(den4, rrep)


def _mid_body(acc_ref, g_ref, be_ref, b_ref,
              wl_ref, wr_ref, aatt_ref, rrep_ref,
              xl_ref, xr_ref, init_ref):
    rrep = rrep_ref[...]
    h = _combine(acc_ref, rrep)
    h = _elu(_ln(h + b_ref[...], g_ref[...], be_ref[...]))
    xl = _mm(h, wl_ref[...])
    xr = _mm(h, wr_ref[...])
    sn, e = _self_terms(xl, xr, aatt_ref[...], rrep)
    _store_init(xl, xr, sn, e, xl_ref, xr_ref, init_ref)


def _post_body(acc_ref, mh_ref, rrep_ref,
               b1_ref, g1_ref, be1_ref, wp_ref, bp_ref, gf_ref, bf_ref,
               out_ref):
    outm = _combine(acc_ref, rrep_ref[...])
    h = _mm(outm, mh_ref[...])
    h = _elu(_ln(h + b1_ref[...], g1_ref[...], be1_ref[...]))
    emb = _mm(h, wp_ref[...]) + bp_ref[...]
    emb = _ln(emb, gf_ref[...], bf_ref[...])
    nrm = jnp.sqrt(jnp.sum(emb * emb, axis=1, keepdims=True))
    out_ref[...] = emb / jnp.maximum(nrm, 1e-12)


def _row_spec(cols):
    return pl.BlockSpec((_BN, cols), lambda i: (i, 0))


def _acc_spec():
    return pl.BlockSpec((_NC, _BN, _ROW), lambda i: (0, i, 0))


def _full_spec(shape):
    return pl.BlockSpec(shape, lambda i: tuple(0 for _ in shape))


_GRID = (_N // _BN,)

_STAGE_OUT = [jax.ShapeDtypeStruct((_N, _ROW), _F32),
              jax.ShapeDtypeStruct((_N, _HC), _F32),
              jax.ShapeDtypeStruct((_NC, _N, _ROW), _F32)]
_STAGE_OUT_SPECS = [_row_spec(_ROW), _row_spec(_HC), _acc_spec()]

_pre_call = pl.pallas_call(
    _pre_body,
    grid=_GRID,
    in_specs=[_row_spec(4), _row_spec(1), _full_spec((17, 12)),
              _full_spec((_IN0, _HC)), _full_spec((_IN0, _HC)),
              _full_spec((_HC, _H)), _full_spec((_H, _HC))],
    out_specs=_STAGE_OUT_SPECS,
    out_shape=_STAGE_OUT,
)

_mid_call = pl.pallas_call(
    _mid_body,
    grid=_GRID,
    in_specs=[_acc_spec(),
              _full_spec((1, _HC)), _full_spec((1, _HC)), _full_spec((1, _HC)),
              _full_spec((_HC, _HC)), _full_spec((_HC, _HC)),
              _full_spec((_HC, _H)), _full_spec((_H, _HC))],
    out_specs=_STAGE_OUT_SPECS,
    out_shape=_STAGE_OUT,
)

_post_call = pl.pallas_call(
    _post_body,
    grid=_GRID,
    in_specs=[_acc_spec(),
              _full_spec((_HC, _C)), _full_spec((_H, _HC)),
              _full_spec((1, _C)), _full_spec((1, _C)), _full_spec((1, _C)),
              _full_spec((_C, _OUT)), _full_spec((1, _OUT)),
              _full_spec((1, _OUT)), _full_spec((1, _OUT))],
    out_specs=[_row_spec(_OUT)],
    out_shape=[jax.ShapeDtypeStruct((_N, _OUT), _F32)],
)


def _att_mat(att):
    return (att[:, :, None] * jnp.eye(_H, dtype=_F32)[:, None, :]).reshape(_HC, _H)


def kernel(x, joint_types, edge_index, emb_table, Wl0, Wr0, att0, b0, g0, be0,
           Wl1, Wr1, att1, b1, g1, be1, Wp, bp, gf, bf):
    jt = joint_types.astype(jnp.int32).reshape(_N, 1)
    src = edge_index[0].astype(jnp.int32)
    dst = edge_index[1].astype(jnp.int32)

    rrep = jnp.repeat(jnp.eye(_H, dtype=_F32), _C, axis=1)
    mh = jnp.tile(jnp.eye(_C, dtype=_F32), (_H, 1)) * (1.0 / _H)
    aatt0 = _att_mat(att0)
    aatt1 = _att_mat(att1)
    attv0 = att0.reshape(_HC)
    attv1 = att1.reshape(_HC)

    xl0, xr0, init0 = _pre_call(x, jt, emb_table, Wl0, Wr0, aatt0, rrep)
    (acc0,) = _sc_call(src, dst, xl0, xr0, attv0, init0)

    xl1, xr1, init1 = _mid_call(acc0,
                                g0.reshape(1, _HC), be0.reshape(1, _HC),
                                b0.reshape(1, _HC), Wl1, Wr1, aatt1, rrep)
    (acc1,) = _sc_call(src, dst, xl1, xr1, attv1, init1)

    (out,) = _post_call(acc1, mh, rrep,
                        b1.reshape(1, _C), g1.reshape(1, _C),
                        be1.reshape(1, _C), Wp, bp.reshape(1, _OUT),
                        gf.reshape(1, _OUT), bf.reshape(1, _OUT))
    return out
